# Initial kernel scaffold; baseline (speedup 1.0000x reference)
import jax
import jax.numpy as jnp
from jax.experimental import pallas as pl


def kernel(x, edge_index, W1, b1, W2, b2, W3, b3, W4, b4, W5, b5, W6, b6, g1, be1, g2, be2, g3, be3, g4, be4, g5, be5):
    raise NotImplementedError("write your pallas kernel here")


# speedup vs baseline: 6.6251x; 6.6251x over previous
import functools

import jax
import jax.numpy as jnp
from jax import lax
from jax.experimental import pallas as pl
from jax.experimental.pallas import tpu as pltpu
from jax.experimental.pallas import tpu_sc as plsc

N = 10000
E = 320000
IN_DIM = 128
HID = 256
OUT_DIM = 128
EPS = 1e-5

_NC = 2
_NS = 16
NPAD = 10112
EPAD = 323584


def _make_agg(col_split):
    dh = 128
    K = 128
    n_workers = _NS if col_split else _NC * _NS
    e_per_tile = EPAD // n_workers
    n_chunks = e_per_tile // K
    rows_per_tile = NPAD // _NS

    mesh = plsc.VectorSubcoreMesh(core_axis_name="c", subcore_axis_name="s")

    @functools.partial(
        pl.kernel,
        out_type=jax.ShapeDtypeStruct((_NC, NPAD, dh), jnp.float32),
        mesh=mesh,
        scratch_types=[
            pltpu.VMEM((K,), jnp.int32),
            pltpu.VMEM((K,), jnp.int32),
            pltpu.VMEM((K, dh), jnp.float32),
            pltpu.VMEM_SHARED((NPAD, dh), jnp.float32),
            pltpu.SemaphoreType.DMA,
        ],
    )
    def agg(src_hbm, dst_hbm, u_hbm, init_hbm, out_hbm,
            src_v, dst_v, rows_v, acc_sh, sem):
        c = lax.axis_index("c")
        s = lax.axis_index("s")
        r0 = s * rows_per_tile
        pltpu.sync_copy(init_hbm.at[c].at[pl.ds(r0, rows_per_tile)],
                        acc_sh.at[pl.ds(r0, rows_per_tile)])
        plsc.subcore_barrier()
        e0 = (s if col_split else c * _NS + s) * e_per_tile

        def body(i, carry):
            off = e0 + i * K
            pltpu.sync_copy(src_hbm.at[pl.ds(off, K)], src_v)
            pltpu.sync_copy(dst_hbm.at[pl.ds(off, K)], dst_v)
            table = u_hbm.at[c] if col_split else u_hbm
            pltpu.async_copy(table.at[src_v], rows_v, sem).wait()
            pltpu.sync_copy(rows_v, acc_sh.at[dst_v], add=True)
            return carry

        lax.fori_loop(0, n_chunks, body, 0)
        plsc.subcore_barrier()
        pltpu.sync_copy(acc_sh.at[pl.ds(r0, rows_per_tile)],
                        out_hbm.at[c].at[pl.ds(r0, rows_per_tile)])

    return agg


def _make_deg():
    K = 64
    e_per_tile = EPAD // (_NC * _NS)
    n_chunks = e_per_tile // K
    rows_per_tile = NPAD // _NS

    mesh = plsc.VectorSubcoreMesh(core_axis_name="c", subcore_axis_name="s")

    @functools.partial(
        pl.kernel,
        out_type=jax.ShapeDtypeStruct((_NC, NPAD, 16), jnp.float32),
        mesh=mesh,
        scratch_types=[
            pltpu.VMEM((K,), jnp.int32),
            pltpu.VMEM((K, 16), jnp.float32),
            pltpu.VMEM_SHARED((NPAD, 16), jnp.float32),
        ],
    )
    def deg(dst_hbm, zeros_hbm, out_hbm, dst_v, ones_v, acc_sh):
        c = lax.axis_index("c")
        s = lax.axis_index("s")
        r0 = s * rows_per_tile
        pltpu.sync_copy(zeros_hbm.at[pl.ds(r0, rows_per_tile)],
                        acc_sh.at[pl.ds(r0, rows_per_tile)])
        one = jnp.ones((16,), jnp.float32)
        for j in range(K):
            ones_v[j] = one
        plsc.subcore_barrier()
        e0 = (c * _NS + s) * e_per_tile

        def body(i, carry):
            off = e0 + i * K
            pltpu.sync_copy(dst_hbm.at[pl.ds(off, K)], dst_v)
            pltpu.sync_copy(ones_v, acc_sh.at[dst_v], add=True)
            return carry

        lax.fori_loop(0, n_chunks, body, 0)
        plsc.subcore_barrier()
        pltpu.sync_copy(acc_sh.at[pl.ds(r0, rows_per_tile)],
                        out_hbm.at[c].at[pl.ds(r0, rows_per_tile)])

    return deg


_AGG_COL = _make_agg(True)
_AGG_EDGE = _make_agg(False)
_DEG = _make_deg()


_B = 632
_NBLK = NPAD // _B


def _prep_body(cnt_ref, x_ref, u_ref, init_ref, dinv_ref):
    deg = cnt_ref[0, :, 0:1] + cnt_ref[1, :, 0:1] + 1.0
    dinv = lax.rsqrt(deg)
    u = x_ref[...] * dinv
    iv = u * dinv
    u_ref[...] = u
    init_ref[0] = iv
    init_ref[1] = jnp.zeros((NPAD, 128), jnp.float32)
    dinv_ref[...] = dinv


def _dot(a, b):
    return jax.lax.dot_general(a, b, (((1,), (0,)), ((), ())),
                               precision=lax.Precision.HIGHEST,
                               preferred_element_type=jnp.float32)


def _make_layer_body(in_sum, pre_w, out_full):
    def body(acc_ref, dinv_ref, g_ref, be_ref, *rest):
        if pre_w:
            wpre_ref, w_ref, u_ref, init_ref = rest
        else:
            w_ref, u_ref, init_ref = rest

        def block_y(i):
            rows = pl.ds(i * _B, _B)
            dinv_b = dinv_ref[rows, :]
            if in_sum:
                y = (acc_ref[0, rows, :] + acc_ref[1, rows, :]) * dinv_b
            else:
                y = jnp.concatenate([acc_ref[0, rows, :],
                                     acc_ref[1, rows, :]], axis=1) * dinv_b
            if pre_w:
                y = _dot(y, wpre_ref[...])
            return y, dinv_b

        def stat_block(i, carry):
            s1, s2 = carry
            y, _ = block_y(i)
            valid = (i * _B + lax.broadcasted_iota(jnp.int32, (_B, 1), 0)) < N
            y = jnp.where(valid, y, 0.0)
            return (s1 + jnp.sum(y, axis=0, keepdims=True),
                    s2 + jnp.sum(y * y, axis=0, keepdims=True))

        width = HID if (pre_w or not in_sum) else 128
        zero_s = jnp.zeros((1, width), jnp.float32)
        s1, s2 = lax.fori_loop(0, _NBLK, stat_block, (zero_s, zero_s))
        mu = s1 * (1.0 / N)
        var = s2 * (1.0 / N) - mu * mu
        scale = lax.rsqrt(var + EPS) * g_ref[...]
        shift = be_ref[...] - mu * scale

        def apply_block(i, carry):
            rows = pl.ds(i * _B, _B)
            y, dinv_b = block_y(i)
            z = jnp.maximum(y * scale + shift, 0.0)
            m = _dot(z, w_ref[...])
            u = m * dinv_b
            iv = u * dinv_b
            if out_full:
                u_ref[rows, :] = u
                init_ref[0, rows, :] = iv
                init_ref[1, rows, :] = jnp.zeros((_B, 128), jnp.float32)
            else:
                u_ref[0, rows, :] = u[:, :128]
                u_ref[1, rows, :] = u[:, 128:]
                init_ref[0, rows, :] = iv[:, :128]
                init_ref[1, rows, :] = iv[:, 128:]
            return carry

        lax.fori_loop(0, _NBLK, apply_block, 0)

    return body


def _final_body(acc_ref, dinv_ref, b_ref, out_ref):
    y = (acc_ref[0, :N, :] + acc_ref[1, :N, :]) * dinv_ref[:N, :]
    out_ref[...] = y + b_ref[...]


def _u_out(split):
    u_shape = (_NC, NPAD, 128) if split else (NPAD, 128)
    return (jax.ShapeDtypeStruct(u_shape, jnp.float32),
            jax.ShapeDtypeStruct((_NC, NPAD, 128), jnp.float32))


_PREP = pl.pallas_call(
    _prep_body,
    out_shape=(*_u_out(False), jax.ShapeDtypeStruct((NPAD, 1), jnp.float32)))

_LAYER_FIRST = pl.pallas_call(
    _make_layer_body(True, True, False), out_shape=_u_out(True))
_LAYER_MID = pl.pallas_call(
    _make_layer_body(False, False, False), out_shape=_u_out(True))
_LAYER_LAST = pl.pallas_call(
    _make_layer_body(False, False, True), out_shape=_u_out(False))
_FINAL = pl.pallas_call(
    _final_body, out_shape=jax.ShapeDtypeStruct((N, OUT_DIM), jnp.float32))


def kernel(x, edge_index, W1, b1, W2, b2, W3, b3, W4, b4, W5, b5, W6, b6,
           g1, be1, g2, be2, g3, be3, g4, be4, g5, be5):
    src = edge_index[0].astype(jnp.int32)
    dst = edge_index[1].astype(jnp.int32)
    pad = EPAD - E
    src_p = jnp.concatenate([src, jnp.zeros((pad,), jnp.int32)])
    dst_p = jnp.concatenate([dst, jnp.full((pad,), N, jnp.int32)])
    zeros16 = jnp.zeros((NPAD, 16), jnp.float32)
    x_p = jnp.concatenate([x, jnp.zeros((NPAD - N, IN_DIM), jnp.float32)])

    cnt = _DEG(dst_p, zeros16)
    u, iv, dinv = _PREP(cnt, x_p)
    a = _AGG_EDGE(src_p, dst_p, u, iv)
    u, iv = _LAYER_FIRST(a, dinv, g1.reshape(1, HID), be1.reshape(1, HID),
                         W1, W2)
    a = _AGG_COL(src_p, dst_p, u, iv)
    u, iv = _LAYER_MID(a, dinv, g2.reshape(1, HID), be2.reshape(1, HID), W3)
    a = _AGG_COL(src_p, dst_p, u, iv)
    u, iv = _LAYER_MID(a, dinv, g3.reshape(1, HID), be3.reshape(1, HID), W4)
    a = _AGG_COL(src_p, dst_p, u, iv)
    u, iv = _LAYER_MID(a, dinv, g4.reshape(1, HID), be4.reshape(1, HID), W5)
    a = _AGG_COL(src_p, dst_p, u, iv)
    u, iv = _LAYER_LAST(a, dinv, g5.reshape(1, HID), be5.reshape(1, HID), W6)
    a = _AGG_EDGE(src_p, dst_p, u, iv)
    out = _FINAL(a, dinv, b6.reshape(1, OUT_DIM))
    return out


# speedup vs baseline: 6.6261x; 1.0001x over previous
import functools

import jax
import jax.numpy as jnp
from jax import lax
from jax.experimental import pallas as pl
from jax.experimental.pallas import tpu as pltpu
from jax.experimental.pallas import tpu_sc as plsc

N = 10000
E = 320000
IN_DIM = 128
HID = 256
OUT_DIM = 128
EPS = 1e-5

_NC = 2
_NS = 16
NPAD = 10112
EPAD = 327680


_GRP = 8


def _make_agg(col_split):
    dh = 128
    K = 128
    n_workers = _NS if col_split else _NC * _NS
    nc = EPAD // (n_workers * K)
    nblk = nc // _GRP
    rows_per_tile = NPAD // _NS

    mesh = plsc.VectorSubcoreMesh(core_axis_name="c", subcore_axis_name="s")

    @functools.partial(
        pl.kernel,
        out_type=jax.ShapeDtypeStruct((_NC, NPAD, dh), jnp.float32),
        mesh=mesh,
        scratch_types=[
            pltpu.VMEM((_GRP, K), jnp.int32),
            pltpu.VMEM((_GRP, K), jnp.int32),
            pltpu.VMEM((2, K, dh), jnp.float32),
            pltpu.VMEM_SHARED((NPAD, dh), jnp.float32),
            [pltpu.SemaphoreType.DMA] * 2,
        ],
    )
    def agg(src_hbm, dst_hbm, u_hbm, init_hbm, out_hbm,
            sidx, didx, rows_v, acc_sh, sems):
        c = lax.axis_index("c")
        s = lax.axis_index("s")
        r0 = s * rows_per_tile
        pltpu.sync_copy(init_hbm.at[c].at[pl.ds(r0, rows_per_tile)],
                        acc_sh.at[pl.ds(r0, rows_per_tile)])
        blk0 = (s if col_split else c * _NS + s) * nblk
        table = u_hbm.at[c] if col_split else u_hbm
        plsc.subcore_barrier()

        def gather(k):
            return pltpu.make_async_copy(table.at[sidx.at[k]],
                                         rows_v.at[k % 2], sems[k % 2])

        def scatter(k):
            return pltpu.make_async_copy(rows_v.at[k % 2],
                                         acc_sh.at[didx.at[k]],
                                         sems[k % 2])

        def block(g, carry):
            pltpu.sync_copy(src_hbm.at[blk0 + g], sidx)
            pltpu.sync_copy(dst_hbm.at[blk0 + g], didx)
            gather(0).start()
            for k in range(1, _GRP):
                if k >= 2:
                    scatter(k - 2).wait()
                gather(k).start()
                gather(k - 1).wait()
                pltpu.async_copy(rows_v.at[(k - 1) % 2],
                                 acc_sh.at[didx.at[k - 1]],
                                 sems[(k - 1) % 2], add=True)
            gather(_GRP - 1).wait()
            scatter(_GRP - 2).wait()
            pltpu.async_copy(rows_v.at[(_GRP - 1) % 2],
                             acc_sh.at[didx.at[_GRP - 1]],
                             sems[(_GRP - 1) % 2], add=True)
            scatter(_GRP - 1).wait()
            return carry

        lax.fori_loop(0, nblk, block, 0)
        plsc.subcore_barrier()
        pltpu.sync_copy(acc_sh.at[pl.ds(r0, rows_per_tile)],
                        out_hbm.at[c].at[pl.ds(r0, rows_per_tile)])

    return agg


def _make_deg():
    K = 128
    nc = EPAD // (_NC * _NS * K)
    nblk = nc // _GRP
    rows_per_tile = NPAD // _NS

    mesh = plsc.VectorSubcoreMesh(core_axis_name="c", subcore_axis_name="s")

    @functools.partial(
        pl.kernel,
        out_type=jax.ShapeDtypeStruct((_NC, NPAD, 16), jnp.float32),
        mesh=mesh,
        scratch_types=[
            pltpu.VMEM((_GRP, K), jnp.int32),
            pltpu.VMEM((K, 16), jnp.float32),
            pltpu.VMEM_SHARED((NPAD, 16), jnp.float32),
            pltpu.SemaphoreType.DMA,
        ],
    )
    def deg(dst_hbm, zeros_hbm, out_hbm, didx, ones_v, acc_sh, sem):
        c = lax.axis_index("c")
        s = lax.axis_index("s")
        r0 = s * rows_per_tile
        pltpu.sync_copy(zeros_hbm.at[pl.ds(r0, rows_per_tile)],
                        acc_sh.at[pl.ds(r0, rows_per_tile)])
        blk0 = (c * _NS + s) * nblk
        one = jnp.ones((16,), jnp.float32)
        for j in range(K):
            ones_v[j] = one
        plsc.subcore_barrier()

        def block(g, carry):
            pltpu.sync_copy(dst_hbm.at[blk0 + g], didx)
            for t in range(_GRP):
                pltpu.async_copy(ones_v, acc_sh.at[didx.at[t]],
                                 sem, add=True)
            for t in range(_GRP):
                pltpu.make_async_copy(ones_v, acc_sh.at[didx.at[t]],
                                      sem).wait()
            return carry

        lax.fori_loop(0, nblk, block, 0)
        plsc.subcore_barrier()
        pltpu.sync_copy(acc_sh.at[pl.ds(r0, rows_per_tile)],
                        out_hbm.at[c].at[pl.ds(r0, rows_per_tile)])

    return deg


_AGG_COL = _make_agg(True)
_AGG_EDGE = _make_agg(False)
_DEG = _make_deg()


_B = 632
_NBLK = NPAD // _B


def _prep_body(cnt_ref, x_ref, u_ref, init_ref, dinv_ref):
    deg = cnt_ref[0, :, 0:1] + cnt_ref[1, :, 0:1] + 1.0
    dinv = lax.rsqrt(deg)
    u = x_ref[...] * dinv
    iv = u * dinv
    u_ref[...] = u
    init_ref[0] = iv
    init_ref[1] = jnp.zeros((NPAD, 128), jnp.float32)
    dinv_ref[...] = dinv


def _dot(a, b):
    return jax.lax.dot_general(a, b, (((1,), (0,)), ((), ())),
                               precision=lax.Precision.HIGHEST,
                               preferred_element_type=jnp.float32)


def _make_layer_body(in_sum, pre_w, out_full):
    def body(acc_ref, dinv_ref, g_ref, be_ref, *rest):
        if pre_w:
            wpre_ref, w_ref, u_ref, init_ref = rest
        else:
            w_ref, u_ref, init_ref = rest

        def block_y(i):
            rows = pl.ds(i * _B, _B)
            dinv_b = dinv_ref[rows, :]
            if in_sum:
                y = (acc_ref[0, rows, :] + acc_ref[1, rows, :]) * dinv_b
            else:
                y = jnp.concatenate([acc_ref[0, rows, :],
                                     acc_ref[1, rows, :]], axis=1) * dinv_b
            if pre_w:
                y = _dot(y, wpre_ref[...])
            return y, dinv_b

        def stat_block(i, carry):
            s1, s2 = carry
            y, _ = block_y(i)
            valid = (i * _B + lax.broadcasted_iota(jnp.int32, (_B, 1), 0)) < N
            y = jnp.where(valid, y, 0.0)
            return (s1 + jnp.sum(y, axis=0, keepdims=True),
                    s2 + jnp.sum(y * y, axis=0, keepdims=True))

        width = HID if (pre_w or not in_sum) else 128
        zero_s = jnp.zeros((1, width), jnp.float32)
        s1, s2 = lax.fori_loop(0, _NBLK, stat_block, (zero_s, zero_s))
        mu = s1 * (1.0 / N)
        var = s2 * (1.0 / N) - mu * mu
        scale = lax.rsqrt(var + EPS) * g_ref[...]
        shift = be_ref[...] - mu * scale

        def apply_block(i, carry):
            rows = pl.ds(i * _B, _B)
            y, dinv_b = block_y(i)
            z = jnp.maximum(y * scale + shift, 0.0)
            m = _dot(z, w_ref[...])
            u = m * dinv_b
            iv = u * dinv_b
            if out_full:
                u_ref[rows, :] = u
                init_ref[0, rows, :] = iv
                init_ref[1, rows, :] = jnp.zeros((_B, 128), jnp.float32)
            else:
                u_ref[0, rows, :] = u[:, :128]
                u_ref[1, rows, :] = u[:, 128:]
                init_ref[0, rows, :] = iv[:, :128]
                init_ref[1, rows, :] = iv[:, 128:]
            return carry

        lax.fori_loop(0, _NBLK, apply_block, 0)

    return body


def _final_body(acc_ref, dinv_ref, b_ref, out_ref):
    y = (acc_ref[0, :N, :] + acc_ref[1, :N, :]) * dinv_ref[:N, :]
    out_ref[...] = y + b_ref[...]


def _u_out(split):
    u_shape = (_NC, NPAD, 128) if split else (NPAD, 128)
    return (jax.ShapeDtypeStruct(u_shape, jnp.float32),
            jax.ShapeDtypeStruct((_NC, NPAD, 128), jnp.float32))


_PREP = pl.pallas_call(
    _prep_body,
    out_shape=(*_u_out(False), jax.ShapeDtypeStruct((NPAD, 1), jnp.float32)))

_LAYER_FIRST = pl.pallas_call(
    _make_layer_body(True, True, False), out_shape=_u_out(True))
_LAYER_MID = pl.pallas_call(
    _make_layer_body(False, False, False), out_shape=_u_out(True))
_LAYER_LAST = pl.pallas_call(
    _make_layer_body(False, False, True), out_shape=_u_out(False))
_FINAL = pl.pallas_call(
    _final_body, out_shape=jax.ShapeDtypeStruct((N, OUT_DIM), jnp.float32))


def kernel(x, edge_index, W1, b1, W2, b2, W3, b3, W4, b4, W5, b5, W6, b6,
           g1, be1, g2, be2, g3, be3, g4, be4, g5, be5):
    src = edge_index[0].astype(jnp.int32)
    dst = edge_index[1].astype(jnp.int32)
    pad = EPAD - E
    src_p = jnp.concatenate([src, jnp.zeros((pad,), jnp.int32)]).reshape(-1, _GRP, 128)
    dst_p = jnp.concatenate([dst, jnp.full((pad,), N, jnp.int32)]).reshape(-1, _GRP, 128)
    zeros16 = jnp.zeros((NPAD, 16), jnp.float32)
    x_p = jnp.concatenate([x, jnp.zeros((NPAD - N, IN_DIM), jnp.float32)])

    cnt = _DEG(dst_p, zeros16)
    u, iv, dinv = _PREP(cnt, x_p)
    a = _AGG_EDGE(src_p, dst_p, u, iv)
    u, iv = _LAYER_FIRST(a, dinv, g1.reshape(1, HID), be1.reshape(1, HID),
                         W1, W2)
    a = _AGG_COL(src_p, dst_p, u, iv)
    u, iv = _LAYER_MID(a, dinv, g2.reshape(1, HID), be2.reshape(1, HID), W3)
    a = _AGG_COL(src_p, dst_p, u, iv)
    u, iv = _LAYER_MID(a, dinv, g3.reshape(1, HID), be3.reshape(1, HID), W4)
    a = _AGG_COL(src_p, dst_p, u, iv)
    u, iv = _LAYER_MID(a, dinv, g4.reshape(1, HID), be4.reshape(1, HID), W5)
    a = _AGG_COL(src_p, dst_p, u, iv)
    u, iv = _LAYER_LAST(a, dinv, g5.reshape(1, HID), be5.reshape(1, HID), W6)
    a = _AGG_EDGE(src_p, dst_p, u, iv)
    out = _FINAL(a, dinv, b6.reshape(1, OUT_DIM))
    return out
